# baseline (device time: 77602 ns/iter reference)
import jax
import jax.numpy as jnp
from jax import lax
from jax.experimental import pallas as pl
from jax.experimental.pallas import tpu as pltpu

N_DEV = 4


def kernel(x, w_mat):
    m_global, k_shard = x.shape
    k_global, n = w_mat.shape
    m_per = m_global // N_DEV

    def body(x_hbm, w_hbm, out_ref, xb_ref, stage_ref, comm_ref, wf32_ref,
             amax_ref, xb_sems, send_sems, recv_sems, w_sems,
             amax_send_sems, amax_recv_sems):
        me = lax.axis_index("i")

        barrier_sem = pltpu.get_barrier_semaphore()
        for off in range(1, N_DEV):
            peer = lax.rem(me + off, N_DEV)
            pl.semaphore_signal(
                barrier_sem, inc=1,
                device_id=(peer,), device_id_type=pl.DeviceIdType.MESH,
            )

        def x_start(d, slot):
            cp = pltpu.make_async_copy(
                x_hbm.at[pl.ds(d * m_per, m_per), :],
                xb_ref.at[slot],
                xb_sems.at[slot],
            )
            cp.start()
            return cp

        chunk_order = [me] + [lax.rem(me + off, N_DEV) for off in (3, 2, 1)]

        def w_start(j, slot):
            d = chunk_order[j]
            cp = pltpu.make_async_copy(
                w_hbm.at[pl.ds(d * k_shard, k_shard), :],
                wf32_ref.at[slot],
                w_sems.at[slot],
            )
            cp.start()
            return cp

        w_copies = {0: w_start(0, 0), 1: w_start(1, 1)}
        x_copies = {1: x_start(lax.rem(me + 1, N_DEV), 0),
                    2: x_start(lax.rem(me + 2, N_DEV), 1)}

        rdmas = []
        for off in range(1, N_DEV):
            slot = (off - 1) % 2
            x_copies[off].wait()
            stage_ref[off - 1] = xb_ref[slot].astype(jnp.bfloat16)
            if off == 1:
                x_copies[3] = x_start(lax.rem(me + 3, N_DEV), 0)
            elif off == 2:
                x_copies[4] = x_start(me, 1)
            if off == 1:
                pl.semaphore_wait(barrier_sem, N_DEV - 1)
            peer = lax.rem(me + off, N_DEV)
            rdma = pltpu.make_async_remote_copy(
                src_ref=stage_ref.at[off - 1],
                dst_ref=comm_ref.at[off - 1],
                send_sem=send_sems.at[off - 1],
                recv_sem=recv_sems.at[off - 1],
                device_id=(peer,),
                device_id_type=pl.DeviceIdType.MESH,
            )
            rdma.start()
            rdmas.append(rdma)
        x_copies[4].wait()
        stage_ref[3] = xb_ref[1].astype(jnp.bfloat16)

        for j in range(N_DEV):
            w_copies[j].wait()
            if j == 0:
                lhs = stage_ref[3]
            else:
                recv = pltpu.make_async_remote_copy(
                    src_ref=comm_ref.at[j - 1],
                    dst_ref=comm_ref.at[j - 1],
                    send_sem=send_sems.at[j - 1],
                    recv_sem=recv_sems.at[j - 1],
                    device_id=(me,),
                    device_id_type=pl.DeviceIdType.MESH,
                )
                recv.wait_recv()
                lhs = comm_ref[j - 1]
            part = jnp.dot(
                lhs, wf32_ref[j % 2].astype(jnp.bfloat16),
                preferred_element_type=jnp.float32,
            )
            if j == 0:
                out_ref[...] = part.astype(jnp.bfloat16)
            else:
                out_ref[...] = (out_ref[...] + part).astype(jnp.bfloat16)
            if j + 2 < N_DEV:
                w_copies[j + 2] = w_start(j + 2, j % 2)

        local_amax = jnp.maximum(jnp.max(out_ref[...].astype(jnp.float32)),
                                 0.0)
        amax_ref[3] = jnp.full((8, 128), local_amax, dtype=jnp.float32)

        amax_rdmas = []
        for off in range(1, N_DEV):
            peer = lax.rem(me + off, N_DEV)
            rdma = pltpu.make_async_remote_copy(
                src_ref=amax_ref.at[3],
                dst_ref=amax_ref.at[off - 1],
                send_sem=amax_send_sems.at[off - 1],
                recv_sem=amax_recv_sems.at[off - 1],
                device_id=(peer,),
                device_id_type=pl.DeviceIdType.MESH,
            )
            rdma.start()
            amax_rdmas.append(rdma)
        for idx in range(N_DEV - 1):
            recv = pltpu.make_async_remote_copy(
                src_ref=amax_ref.at[idx],
                dst_ref=amax_ref.at[idx],
                send_sem=amax_send_sems.at[idx],
                recv_sem=amax_recv_sems.at[idx],
                device_id=(me,),
                device_id_type=pl.DeviceIdType.MESH,
            )
            recv.wait_recv()

        gmax = jnp.max(amax_ref[...])
        scale = gmax / 127.0
        y = jnp.maximum(out_ref[...].astype(jnp.float32), 0.0)
        q = jnp.clip(jnp.round(y / scale), -127.0, 127.0)
        out_ref[...] = (q * scale).astype(jnp.bfloat16)

        for rdma in rdmas:
            rdma.wait_send()
        for rdma in amax_rdmas:
            rdma.wait_send()

    return pl.pallas_call(
        body,
        out_shape=jax.ShapeDtypeStruct((m_per, n), jnp.bfloat16),
        in_specs=[
            pl.BlockSpec(memory_space=pl.ANY),
            pl.BlockSpec(memory_space=pl.ANY),
        ],
        out_specs=pl.BlockSpec(memory_space=pltpu.VMEM),
        scratch_shapes=[
            pltpu.VMEM((2, m_per, k_shard), jnp.float32),
            pltpu.VMEM((N_DEV, m_per, k_shard), jnp.bfloat16),
            pltpu.VMEM((N_DEV - 1, m_per, k_shard), jnp.bfloat16),
            pltpu.VMEM((2, k_shard, n), jnp.float32),
            pltpu.VMEM((N_DEV, 8, 128), jnp.float32),
            pltpu.SemaphoreType.DMA((2,)),
            pltpu.SemaphoreType.DMA((N_DEV - 1,)),
            pltpu.SemaphoreType.DMA((N_DEV - 1,)),
            pltpu.SemaphoreType.DMA((2,)),
            pltpu.SemaphoreType.DMA((N_DEV - 1,)),
            pltpu.SemaphoreType.DMA((N_DEV - 1,)),
        ],
        compiler_params=pltpu.CompilerParams(
            collective_id=0,
            vmem_limit_bytes=100 * 1024 * 1024,
        ),
    )(x, w_mat)


# device time: 76285 ns/iter; 1.0173x vs baseline; 1.0173x over previous
import jax
import jax.numpy as jnp
from jax import lax
from jax.experimental import pallas as pl
from jax.experimental.pallas import tpu as pltpu

N_DEV = 4


def kernel(x, w_mat):
    m_global, k_shard = x.shape
    k_global, n = w_mat.shape
    m_per = m_global // N_DEV

    def body(x_hbm, w_hbm, out_ref, xb_ref, stage_ref, comm_ref, wf32_ref,
             wb_ref, amax_ref, xb_sems, send_sems, recv_sems, w_sems,
             amax_send_sems, amax_recv_sems):
        me = lax.axis_index("i")

        barrier_sem = pltpu.get_barrier_semaphore()
        for off in range(1, N_DEV):
            peer = lax.rem(me + off, N_DEV)
            pl.semaphore_signal(
                barrier_sem, inc=1,
                device_id=(peer,), device_id_type=pl.DeviceIdType.MESH,
            )

        def x_start(d, slot):
            cp = pltpu.make_async_copy(
                x_hbm.at[pl.ds(d * m_per, m_per), :],
                xb_ref.at[slot],
                xb_sems.at[slot],
            )
            cp.start()
            return cp

        chunk_order = [me] + [lax.rem(me + off, N_DEV) for off in (3, 2, 1)]

        def w_start(j):
            d = chunk_order[j]
            cp = pltpu.make_async_copy(
                w_hbm.at[pl.ds(d * k_shard, k_shard), :],
                wf32_ref,
                w_sems,
            )
            cp.start()
            return cp

        w_copies = {0: w_start(0)}
        x_copies = {1: x_start(lax.rem(me + 1, N_DEV), 0),
                    2: x_start(lax.rem(me + 2, N_DEV), 1)}

        rdmas = []
        for off in range(1, N_DEV):
            slot = (off - 1) % 2
            x_copies[off].wait()
            stage_ref[off - 1] = xb_ref[slot].astype(jnp.bfloat16)
            if off == 1:
                x_copies[3] = x_start(lax.rem(me + 3, N_DEV), 0)
            elif off == 2:
                x_copies[4] = x_start(me, 1)
            if off == 1:
                pl.semaphore_wait(barrier_sem, N_DEV - 1)
            peer = lax.rem(me + off, N_DEV)
            rdma = pltpu.make_async_remote_copy(
                src_ref=stage_ref.at[off - 1],
                dst_ref=comm_ref.at[off - 1],
                send_sem=send_sems.at[off - 1],
                recv_sem=recv_sems.at[off - 1],
                device_id=(peer,),
                device_id_type=pl.DeviceIdType.MESH,
            )
            rdma.start()
            rdmas.append(rdma)
        x_copies[4].wait()
        stage_ref[3] = xb_ref[1].astype(jnp.bfloat16)

        wb_slot = [0, 1, 2, 0]

        def wb_fill(j):
            w_copies[j].wait()
            wb_ref[wb_slot[j]] = wf32_ref[...].astype(jnp.bfloat16)
            if j + 1 < N_DEV:
                w_copies[j + 1] = w_start(j + 1)

        wb_fill(0)
        part = jnp.dot(stage_ref[3], wb_ref[0],
                       preferred_element_type=jnp.float32)
        out_ref[...] = part.astype(jnp.bfloat16)

        for j in range(1, N_DEV):
            wb_fill(j)

        for j in range(1, N_DEV):
            recv = pltpu.make_async_remote_copy(
                src_ref=comm_ref.at[j - 1],
                dst_ref=comm_ref.at[j - 1],
                send_sem=send_sems.at[j - 1],
                recv_sem=recv_sems.at[j - 1],
                device_id=(me,),
                device_id_type=pl.DeviceIdType.MESH,
            )
            recv.wait_recv()
            part = jnp.dot(comm_ref[j - 1], wb_ref[wb_slot[j]],
                           preferred_element_type=jnp.float32)
            out_ref[...] = (out_ref[...] + part).astype(jnp.bfloat16)

        local_amax = jnp.maximum(jnp.max(out_ref[...].astype(jnp.float32)),
                                 0.0)
        amax_ref[3] = jnp.full((8, 128), local_amax, dtype=jnp.float32)

        amax_rdmas = []
        for off in range(1, N_DEV):
            peer = lax.rem(me + off, N_DEV)
            rdma = pltpu.make_async_remote_copy(
                src_ref=amax_ref.at[3],
                dst_ref=amax_ref.at[off - 1],
                send_sem=amax_send_sems.at[off - 1],
                recv_sem=amax_recv_sems.at[off - 1],
                device_id=(peer,),
                device_id_type=pl.DeviceIdType.MESH,
            )
            rdma.start()
            amax_rdmas.append(rdma)
        for idx in range(N_DEV - 1):
            recv = pltpu.make_async_remote_copy(
                src_ref=amax_ref.at[idx],
                dst_ref=amax_ref.at[idx],
                send_sem=amax_send_sems.at[idx],
                recv_sem=amax_recv_sems.at[idx],
                device_id=(me,),
                device_id_type=pl.DeviceIdType.MESH,
            )
            recv.wait_recv()

        gmax = jnp.max(amax_ref[...])
        scale = gmax / 127.0
        y = jnp.maximum(out_ref[...].astype(jnp.float32), 0.0)
        q = jnp.clip(jnp.round(y / scale), -127.0, 127.0)
        out_ref[...] = (q * scale).astype(jnp.bfloat16)

        for rdma in rdmas:
            rdma.wait_send()
        for rdma in amax_rdmas:
            rdma.wait_send()

    return pl.pallas_call(
        body,
        out_shape=jax.ShapeDtypeStruct((m_per, n), jnp.bfloat16),
        in_specs=[
            pl.BlockSpec(memory_space=pl.ANY),
            pl.BlockSpec(memory_space=pl.ANY),
        ],
        out_specs=pl.BlockSpec(memory_space=pltpu.VMEM),
        scratch_shapes=[
            pltpu.VMEM((2, m_per, k_shard), jnp.float32),
            pltpu.VMEM((N_DEV, m_per, k_shard), jnp.bfloat16),
            pltpu.VMEM((N_DEV - 1, m_per, k_shard), jnp.bfloat16),
            pltpu.VMEM((k_shard, n), jnp.float32),
            pltpu.VMEM((3, k_shard, n), jnp.bfloat16),
            pltpu.VMEM((N_DEV, 8, 128), jnp.float32),
            pltpu.SemaphoreType.DMA((2,)),
            pltpu.SemaphoreType.DMA((N_DEV - 1,)),
            pltpu.SemaphoreType.DMA((N_DEV - 1,)),
            pltpu.SemaphoreType.DMA,
            pltpu.SemaphoreType.DMA((N_DEV - 1,)),
            pltpu.SemaphoreType.DMA((N_DEV - 1,)),
        ],
        compiler_params=pltpu.CompilerParams(
            collective_id=0,
            vmem_limit_bytes=100 * 1024 * 1024,
        ),
    )(x, w_mat)


# device time: 68231 ns/iter; 1.1373x vs baseline; 1.1180x over previous
import jax
import jax.numpy as jnp
from jax import lax
from jax.experimental import pallas as pl
from jax.experimental.pallas import tpu as pltpu

N_DEV = 4


def kernel(x, w_mat):
    m_global, k_shard = x.shape
    k_global, n = w_mat.shape
    m_per = m_global // N_DEV

    def body(x_hbm, w_hbm, out_ref, xb_ref, stage_ref, own_ref, comm_ref,
             sscale_ref, rscale_ref, wf32_ref, wb_ref, amax_ref,
             xb_sems, send_sems, recv_sems, ssc_sems, rsc_sems, w_sems,
             amax_send_sems, amax_recv_sems):
        me = lax.axis_index("i")

        barrier_sem = pltpu.get_barrier_semaphore()
        for off in range(1, N_DEV):
            peer = lax.rem(me + off, N_DEV)
            pl.semaphore_signal(
                barrier_sem, inc=1,
                device_id=(peer,), device_id_type=pl.DeviceIdType.MESH,
            )

        def x_start(d, slot):
            cp = pltpu.make_async_copy(
                x_hbm.at[pl.ds(d * m_per, m_per), :],
                xb_ref.at[slot],
                xb_sems.at[slot],
            )
            cp.start()
            return cp

        chunk_order = [me] + [lax.rem(me + off, N_DEV) for off in (3, 2, 1)]

        def w_start(j):
            d = chunk_order[j]
            cp = pltpu.make_async_copy(
                w_hbm.at[pl.ds(d * k_shard, k_shard), :],
                wf32_ref,
                w_sems,
            )
            cp.start()
            return cp

        w_copies = {0: w_start(0)}
        x_copies = {1: x_start(lax.rem(me + 1, N_DEV), 0),
                    2: x_start(lax.rem(me + 2, N_DEV), 1)}

        rdmas = []
        for off in range(1, N_DEV):
            slot = (off - 1) % 2
            x_copies[off].wait()
            xb = xb_ref[slot]
            s = jnp.max(jnp.abs(xb), axis=1, keepdims=True) / 127.0
            sscale_ref[off - 1] = s
            stage_ref[off - 1] = jnp.clip(
                jnp.round(xb / s), -127.0, 127.0
            ).astype(jnp.int8)
            if off == 1:
                x_copies[3] = x_start(lax.rem(me + 3, N_DEV), 0)
            elif off == 2:
                x_copies[4] = x_start(me, 1)
            if off == 1:
                pl.semaphore_wait(barrier_sem, N_DEV - 1)
            peer = lax.rem(me + off, N_DEV)
            sc = pltpu.make_async_remote_copy(
                src_ref=sscale_ref.at[off - 1],
                dst_ref=rscale_ref.at[off - 1],
                send_sem=ssc_sems.at[off - 1],
                recv_sem=rsc_sems.at[off - 1],
                device_id=(peer,),
                device_id_type=pl.DeviceIdType.MESH,
            )
            sc.start()
            rdma = pltpu.make_async_remote_copy(
                src_ref=stage_ref.at[off - 1],
                dst_ref=comm_ref.at[off - 1],
                send_sem=send_sems.at[off - 1],
                recv_sem=recv_sems.at[off - 1],
                device_id=(peer,),
                device_id_type=pl.DeviceIdType.MESH,
            )
            rdma.start()
            rdmas.extend([sc, rdma])
        x_copies[4].wait()
        own_ref[...] = xb_ref[1].astype(jnp.bfloat16)

        wb_slot = [0, 1, 2, 0]

        def wb_fill(j):
            w_copies[j].wait()
            wb_ref[wb_slot[j]] = wf32_ref[...].astype(jnp.bfloat16)
            if j + 1 < N_DEV:
                w_copies[j + 1] = w_start(j + 1)

        wb_fill(0)
        out_ref[...] = jnp.dot(own_ref[...], wb_ref[0],
                               preferred_element_type=jnp.float32)

        for j in range(1, N_DEV):
            wb_fill(j)

        for j in range(1, N_DEV):
            for sems, ref in ((rsc_sems, rscale_ref), (recv_sems, comm_ref)):
                recv = pltpu.make_async_remote_copy(
                    src_ref=ref.at[j - 1],
                    dst_ref=ref.at[j - 1],
                    send_sem=sems.at[j - 1],
                    recv_sem=sems.at[j - 1],
                    device_id=(me,),
                    device_id_type=pl.DeviceIdType.MESH,
                )
                recv.wait_recv()
            lhs = (comm_ref[j - 1].astype(jnp.float32)
                   * rscale_ref[j - 1]).astype(jnp.bfloat16)
            part = jnp.dot(lhs, wb_ref[wb_slot[j]],
                           preferred_element_type=jnp.float32)
            out_ref[...] = out_ref[...] + part

        local_amax = jnp.maximum(jnp.max(out_ref[...]), 0.0)
        amax_ref[3] = jnp.full((8, 128), local_amax, dtype=jnp.float32)

        amax_rdmas = []
        for off in range(1, N_DEV):
            peer = lax.rem(me + off, N_DEV)
            rdma = pltpu.make_async_remote_copy(
                src_ref=amax_ref.at[3],
                dst_ref=amax_ref.at[off - 1],
                send_sem=amax_send_sems.at[off - 1],
                recv_sem=amax_recv_sems.at[off - 1],
                device_id=(peer,),
                device_id_type=pl.DeviceIdType.MESH,
            )
            rdma.start()
            amax_rdmas.append(rdma)
        for idx in range(N_DEV - 1):
            recv = pltpu.make_async_remote_copy(
                src_ref=amax_ref.at[idx],
                dst_ref=amax_ref.at[idx],
                send_sem=amax_send_sems.at[idx],
                recv_sem=amax_recv_sems.at[idx],
                device_id=(me,),
                device_id_type=pl.DeviceIdType.MESH,
            )
            recv.wait_recv()

        gmax = jnp.max(amax_ref[...])
        scale = gmax / 127.0
        y = jnp.maximum(out_ref[...], 0.0)
        q = jnp.clip(jnp.round(y / scale), -127.0, 127.0)
        out_ref[...] = q * scale

        for rdma in rdmas:
            rdma.wait_send()
        for rdma in amax_rdmas:
            rdma.wait_send()

    return pl.pallas_call(
        body,
        out_shape=jax.ShapeDtypeStruct((m_per, n), jnp.float32),
        in_specs=[
            pl.BlockSpec(memory_space=pl.ANY),
            pl.BlockSpec(memory_space=pl.ANY),
        ],
        out_specs=pl.BlockSpec(memory_space=pltpu.VMEM),
        scratch_shapes=[
            pltpu.VMEM((2, m_per, k_shard), jnp.float32),
            pltpu.VMEM((N_DEV - 1, m_per, k_shard), jnp.int8),
            pltpu.VMEM((m_per, k_shard), jnp.bfloat16),
            pltpu.VMEM((N_DEV - 1, m_per, k_shard), jnp.int8),
            pltpu.VMEM((N_DEV - 1, m_per, 1), jnp.float32),
            pltpu.VMEM((N_DEV - 1, m_per, 1), jnp.float32),
            pltpu.VMEM((k_shard, n), jnp.float32),
            pltpu.VMEM((3, k_shard, n), jnp.bfloat16),
            pltpu.VMEM((N_DEV, 8, 128), jnp.float32),
            pltpu.SemaphoreType.DMA((2,)),
            pltpu.SemaphoreType.DMA((N_DEV - 1,)),
            pltpu.SemaphoreType.DMA((N_DEV - 1,)),
            pltpu.SemaphoreType.DMA((N_DEV - 1,)),
            pltpu.SemaphoreType.DMA((N_DEV - 1,)),
            pltpu.SemaphoreType.DMA,
            pltpu.SemaphoreType.DMA((N_DEV - 1,)),
            pltpu.SemaphoreType.DMA((N_DEV - 1,)),
        ],
        compiler_params=pltpu.CompilerParams(
            collective_id=0,
            vmem_limit_bytes=100 * 1024 * 1024,
        ),
    )(x, w_mat)


# device time: 67330 ns/iter; 1.1526x vs baseline; 1.0134x over previous
import jax
import jax.numpy as jnp
from jax import lax
from jax.experimental import pallas as pl
from jax.experimental.pallas import tpu as pltpu

N_DEV = 4


def kernel(x, w_mat):
    m_global, k_shard = x.shape
    k_global, n = w_mat.shape
    m_per = m_global // N_DEV

    def body(x_hbm, w_hbm, out_ref, xb_ref, stage_ref, own_ref, comm_ref,
             sscale_ref, rscale_ref, wf32_ref, wb_ref, amax_ref,
             xb_sems, send_sems, recv_sems, ssc_sems, rsc_sems, w_sems,
             amax_send_sems, amax_recv_sems):
        me = lax.axis_index("i")

        barrier_sem = pltpu.get_barrier_semaphore()
        for off in range(1, N_DEV):
            peer = lax.rem(me + off, N_DEV)
            pl.semaphore_signal(
                barrier_sem, inc=1,
                device_id=(peer,), device_id_type=pl.DeviceIdType.MESH,
            )

        def x_start(d, slot):
            cp = pltpu.make_async_copy(
                x_hbm.at[pl.ds(d * m_per, m_per), :],
                xb_ref.at[slot],
                xb_sems.at[slot],
            )
            cp.start()
            return cp

        chunk_order = [me] + [lax.rem(me + off, N_DEV) for off in (3, 2, 1)]

        def w_start(j):
            d = chunk_order[j]
            cp = pltpu.make_async_copy(
                w_hbm.at[pl.ds(d * k_shard, k_shard), :],
                wf32_ref,
                w_sems,
            )
            cp.start()
            return cp

        w_copies = {0: w_start(0)}
        x_copies = {1: x_start(lax.rem(me + 1, N_DEV), 0),
                    2: x_start(lax.rem(me + 2, N_DEV), 1)}

        rdmas = []
        for off in range(1, N_DEV):
            slot = (off - 1) % 2
            x_copies[off].wait()
            xb = xb_ref[slot]
            rmax = jnp.max(jnp.abs(xb), axis=1, keepdims=True)
            sscale_ref[off - 1] = rmax / 127.0
            stage_ref[off - 1] = jnp.round(xb * (127.0 / rmax)).astype(
                jnp.int8
            )
            if off == 1:
                x_copies[3] = x_start(lax.rem(me + 3, N_DEV), 0)
            elif off == 2:
                x_copies[4] = x_start(me, 1)
            if off == 1:
                pl.semaphore_wait(barrier_sem, N_DEV - 1)
            peer = lax.rem(me + off, N_DEV)
            sc = pltpu.make_async_remote_copy(
                src_ref=sscale_ref.at[off - 1],
                dst_ref=rscale_ref.at[off - 1],
                send_sem=ssc_sems.at[off - 1],
                recv_sem=rsc_sems.at[off - 1],
                device_id=(peer,),
                device_id_type=pl.DeviceIdType.MESH,
            )
            sc.start()
            rdma = pltpu.make_async_remote_copy(
                src_ref=stage_ref.at[off - 1],
                dst_ref=comm_ref.at[off - 1],
                send_sem=send_sems.at[off - 1],
                recv_sem=recv_sems.at[off - 1],
                device_id=(peer,),
                device_id_type=pl.DeviceIdType.MESH,
            )
            rdma.start()
            rdmas.extend([sc, rdma])
        x_copies[4].wait()
        own_ref[...] = xb_ref[1].astype(jnp.bfloat16)

        wb_slot = [0, 1, 2, 0]

        def wb_fill(j):
            w_copies[j].wait()
            wb_ref[wb_slot[j]] = wf32_ref[...].astype(jnp.bfloat16)
            if j + 1 < N_DEV:
                w_copies[j + 1] = w_start(j + 1)

        wb_fill(0)
        out_ref[...] = jnp.dot(own_ref[...], wb_ref[0],
                               preferred_element_type=jnp.float32)

        for j in range(1, N_DEV):
            wb_fill(j)

        for j in range(1, N_DEV):
            for sems, ref in ((rsc_sems, rscale_ref), (recv_sems, comm_ref)):
                recv = pltpu.make_async_remote_copy(
                    src_ref=ref.at[j - 1],
                    dst_ref=ref.at[j - 1],
                    send_sem=sems.at[j - 1],
                    recv_sem=sems.at[j - 1],
                    device_id=(me,),
                    device_id_type=pl.DeviceIdType.MESH,
                )
                recv.wait_recv()
            lhs = comm_ref[j - 1].astype(jnp.bfloat16)
            part = jnp.dot(lhs, wb_ref[wb_slot[j]],
                           preferred_element_type=jnp.float32)
            out_ref[...] = out_ref[...] + part * rscale_ref[j - 1]

        local_amax = jnp.maximum(jnp.max(out_ref[...]), 0.0)
        amax_ref[3] = jnp.full((8, 128), local_amax, dtype=jnp.float32)

        amax_rdmas = []
        for off in range(1, N_DEV):
            peer = lax.rem(me + off, N_DEV)
            rdma = pltpu.make_async_remote_copy(
                src_ref=amax_ref.at[3],
                dst_ref=amax_ref.at[off - 1],
                send_sem=amax_send_sems.at[off - 1],
                recv_sem=amax_recv_sems.at[off - 1],
                device_id=(peer,),
                device_id_type=pl.DeviceIdType.MESH,
            )
            rdma.start()
            amax_rdmas.append(rdma)
        for idx in range(N_DEV - 1):
            recv = pltpu.make_async_remote_copy(
                src_ref=amax_ref.at[idx],
                dst_ref=amax_ref.at[idx],
                send_sem=amax_send_sems.at[idx],
                recv_sem=amax_recv_sems.at[idx],
                device_id=(me,),
                device_id_type=pl.DeviceIdType.MESH,
            )
            recv.wait_recv()

        gmax = jnp.max(amax_ref[...])
        scale = gmax / 127.0
        y = jnp.maximum(out_ref[...], 0.0)
        q = jnp.clip(jnp.round(y / scale), -127.0, 127.0)
        out_ref[...] = q * scale

        for rdma in rdmas:
            rdma.wait_send()
        for rdma in amax_rdmas:
            rdma.wait_send()

    return pl.pallas_call(
        body,
        out_shape=jax.ShapeDtypeStruct((m_per, n), jnp.float32),
        in_specs=[
            pl.BlockSpec(memory_space=pl.ANY),
            pl.BlockSpec(memory_space=pl.ANY),
        ],
        out_specs=pl.BlockSpec(memory_space=pltpu.VMEM),
        scratch_shapes=[
            pltpu.VMEM((2, m_per, k_shard), jnp.float32),
            pltpu.VMEM((N_DEV - 1, m_per, k_shard), jnp.int8),
            pltpu.VMEM((m_per, k_shard), jnp.bfloat16),
            pltpu.VMEM((N_DEV - 1, m_per, k_shard), jnp.int8),
            pltpu.VMEM((N_DEV - 1, m_per, 1), jnp.float32),
            pltpu.VMEM((N_DEV - 1, m_per, 1), jnp.float32),
            pltpu.VMEM((k_shard, n), jnp.float32),
            pltpu.VMEM((3, k_shard, n), jnp.bfloat16),
            pltpu.VMEM((N_DEV, 8, 128), jnp.float32),
            pltpu.SemaphoreType.DMA((2,)),
            pltpu.SemaphoreType.DMA((N_DEV - 1,)),
            pltpu.SemaphoreType.DMA((N_DEV - 1,)),
            pltpu.SemaphoreType.DMA((N_DEV - 1,)),
            pltpu.SemaphoreType.DMA((N_DEV - 1,)),
            pltpu.SemaphoreType.DMA,
            pltpu.SemaphoreType.DMA((N_DEV - 1,)),
            pltpu.SemaphoreType.DMA((N_DEV - 1,)),
        ],
        compiler_params=pltpu.CompilerParams(
            collective_id=0,
            vmem_limit_bytes=100 * 1024 * 1024,
        ),
    )(x, w_mat)
